# baseline (speedup 1.0000x reference)
import functools
import math

import jax
import jax.numpy as jnp
from jax import lax
from jax.experimental import pallas as pl
from jax.experimental.pallas import tpu as pltpu
from jax.experimental.pallas import tpu_sc as plsc

_NC = 2
_NS = 16
_NW = _NC * _NS
_LANES = 16


def _embed_lookup(idx, table, chunk):
    n, = idx.shape
    _, d = table.shape
    b_per_w = n // _NW
    nstep = b_per_w // chunk
    scale = math.sqrt(d)

    mesh = plsc.VectorSubcoreMesh(core_axis_name="c", subcore_axis_name="s")

    @functools.partial(
        pl.kernel,
        mesh=mesh,
        out_type=jax.ShapeDtypeStruct((n, d), jnp.float32),
        scratch_types=[
            pltpu.VMEM((chunk,), jnp.int32),
            pltpu.VMEM((chunk, d), jnp.float32),
            pltpu.SemaphoreType.DMA,
        ],
        compiler_params=pltpu.CompilerParams(use_tc_tiling_on_sc=False),
    )
    def run(idx_hbm, table_hbm, out_hbm, idx_v, rows_v, sem):
        wid = lax.axis_index("s") * _NC + lax.axis_index("c")
        base = wid * b_per_w

        @pl.loop(0, nstep)
        def _step(t):
            off = base + t * chunk
            pltpu.sync_copy(idx_hbm.at[pl.ds(off, chunk)], idx_v)
            pltpu.async_copy(table_hbm.at[idx_v], rows_v, sem).wait()

            @pl.loop(0, chunk)
            def _scale(i):
                for k in range(d // _LANES):
                    sl = pl.ds(k * _LANES, _LANES)
                    rows_v[i, sl] = rows_v[i, sl] * scale

            pltpu.sync_copy(rows_v, out_hbm.at[pl.ds(off, chunk)])

    return run(idx, table)


def kernel(x, table):
    b, h = x.shape
    _, d = table.shape
    idx = x.reshape(b * h).astype(jnp.int32)
    out = _embed_lookup(idx, table, chunk=512)
    return out.reshape(b, h, d)


# speedup vs baseline: 1.1371x; 1.1371x over previous
import functools
import math

import jax
import jax.numpy as jnp
from jax import lax
from jax.experimental import pallas as pl
from jax.experimental.pallas import tpu as pltpu
from jax.experimental.pallas import tpu_sc as plsc

_NC = 2
_NS = 16
_NW = _NC * _NS
_LANES = 16
_NBUF = 2


def _embed_lookup(idx, table, chunk):
    n, = idx.shape
    _, d = table.shape
    b_per_w = n // _NW
    nstep = b_per_w // chunk
    assert nstep % _NBUF == 0
    scale = math.sqrt(d)

    mesh = plsc.VectorSubcoreMesh(core_axis_name="c", subcore_axis_name="s")

    @functools.partial(
        pl.kernel,
        mesh=mesh,
        out_type=jax.ShapeDtypeStruct((n, d), jnp.float32),
        scratch_types=[
            [pltpu.VMEM((chunk,), jnp.int32) for _ in range(_NBUF)],
            [pltpu.VMEM((chunk, d), jnp.float32) for _ in range(_NBUF)],
            [pltpu.SemaphoreType.DMA for _ in range(_NBUF)],
            [pltpu.SemaphoreType.DMA for _ in range(_NBUF)],
        ],
        compiler_params=pltpu.CompilerParams(use_tc_tiling_on_sc=False),
    )
    def run(idx_hbm, table_hbm, out_hbm, idx_v, rows_v, gsem, isem):
        wid = lax.axis_index("s") * _NC + lax.axis_index("c")
        base = wid * b_per_w

        for b in range(_NBUF):
            pltpu.sync_copy(idx_hbm.at[pl.ds(base + b * chunk, chunk)],
                            idx_v[b])
            pltpu.async_copy(table_hbm.at[idx_v[b]], rows_v[b], gsem[b])

        def do_chunk(b, off, nxt):
            pltpu.make_async_copy(table_hbm.at[idx_v[b]], rows_v[b],
                                  gsem[b]).wait()
            if nxt is not None:
                pltpu.async_copy(idx_hbm.at[pl.ds(nxt, chunk)],
                                 idx_v[b], isem[b])

            @pl.loop(0, chunk, unroll=4)
            def _scale(i):
                for k in range(d // _LANES):
                    sl = pl.ds(k * _LANES, _LANES)
                    rows_v[b][i, sl] = rows_v[b][i, sl] * scale

            pltpu.sync_copy(rows_v[b], out_hbm.at[pl.ds(off, chunk)])
            if nxt is not None:
                pltpu.make_async_copy(idx_hbm.at[pl.ds(nxt, chunk)],
                                      idx_v[b], isem[b]).wait()
                pltpu.async_copy(table_hbm.at[idx_v[b]], rows_v[b],
                                 gsem[b])

        @pl.loop(0, nstep - _NBUF, step=_NBUF)
        def _step(t0):
            for b in range(_NBUF):
                off = base + (t0 + b) * chunk
                do_chunk(b, off, off + _NBUF * chunk)

        for b in range(_NBUF):
            do_chunk(b, base + (nstep - _NBUF + b) * chunk, None)

    return run(idx, table)


def kernel(x, table):
    b, h = x.shape
    _, d = table.shape
    idx = x.reshape(b * h).astype(jnp.int32)
    out = _embed_lookup(idx, table, chunk=512)
    return out.reshape(b, h, d)
